# baseline (speedup 1.0000x reference)
import functools

import jax
import jax.numpy as jnp
from jax import lax
from jax.experimental import pallas as pl
from jax.experimental.pallas import tpu as pltpu
from jax.experimental.pallas import tpu_sc as plsc

B = 128
D_MODEL = 1024
VOCAB = 1000
VPAD = 1024
TOP_P = 0.9
NEG = -1e30

NV = VPAD // 16
N_TILES = 32
ROWS = B // N_TILES


def _tc_softmax_body(x_ref, w_ref, b_ref, p_ref):
    logits = jnp.dot(x_ref[...], w_ref[...],
                     preferred_element_type=jnp.float32)
    logits = logits + b_ref[...]
    m = jnp.max(logits, axis=1, keepdims=True)
    e = jnp.exp(logits - m)
    s = jnp.sum(e, axis=1, keepdims=True)
    p_ref[...] = e * (1.0 / s)


def _sort16(x, desc):
    if desc:
        return plsc.sort_key_val(x, x, descending=True)[0]
    return jnp.sort(x)


def _insert_bit(u, pos, bitval):
    hi = (u >> pos) << (pos + 1)
    lo = u & ((1 << pos) - 1)
    return hi | (bitval << pos) | lo


def _sc_sort_body(p_hbm, out_hbm, *rows):
    wid = lax.axis_index("s") * 2 + lax.axis_index("c")
    base = wid * ROWS
    for r in range(ROWS):
        pltpu.sync_copy(p_hbm.at[base + r], rows[r])

    for desc in (True, False):
        def init_body(u, desc=desc):
            off = _insert_bit(u, 0, 0 if desc else 1) * 16
            for r in range(ROWS):
                rows[r][pl.ds(off, 16)] = _sort16(rows[r][pl.ds(off, 16)],
                                                  desc)
        plsc.parallel_loop(0, NV // 2, unroll=2)(init_body)

    for K in (2, 4, 8, 16, 32, 64):
        lg_k = K.bit_length() - 1
        dirs = (True,) if K == NV else (True, False)
        n_pair = NV // 2 if K == NV else NV // 4
        J = K // 2
        while J >= 2:
            lg_j = J.bit_length() - 1
            for desc in dirs:
                def cross_body(u, J=J, lg_j=lg_j, lg_k=lg_k, desc=desc):
                    q = _insert_bit(u, lg_k - 1, 0 if desc else 1)
                    i = _insert_bit(q, lg_j, 0)
                    off = i * 16
                    off2 = (i + J) * 16
                    for r in range(ROWS):
                        a = rows[r][pl.ds(off, 16)]
                        b2 = rows[r][pl.ds(off2, 16)]
                        mx = jnp.maximum(a, b2)
                        mn = jnp.minimum(a, b2)
                        if desc:
                            rows[r][pl.ds(off, 16)] = mx
                            rows[r][pl.ds(off2, 16)] = mn
                        else:
                            rows[r][pl.ds(off, 16)] = mn
                            rows[r][pl.ds(off2, 16)] = mx
                plsc.parallel_loop(0, n_pair, unroll=4)(cross_body)
            J //= 2

        for desc in dirs:
            def fuse_body(u, lg_k=lg_k, desc=desc):
                t = _insert_bit(u, lg_k - 1, 0 if desc else 1)
                off = t * 32
                off2 = off + 16
                for r in range(ROWS):
                    a = rows[r][pl.ds(off, 16)]
                    b2 = rows[r][pl.ds(off2, 16)]
                    mx = jnp.maximum(a, b2)
                    mn = jnp.minimum(a, b2)
                    if desc:
                        rows[r][pl.ds(off, 16)] = _sort16(mx, True)
                        rows[r][pl.ds(off2, 16)] = _sort16(mn, True)
                    else:
                        rows[r][pl.ds(off, 16)] = _sort16(mn, False)
                        rows[r][pl.ds(off2, 16)] = _sort16(mx, False)
            plsc.parallel_loop(0, n_pair, unroll=2)(fuse_body)

    for r in range(ROWS):
        pltpu.sync_copy(rows[r], out_hbm.at[base + r])


def _tc_finish_body(p_ref, ps_ref, eg_ref, out_ref):
    ps = ps_ref[...]
    p = p_ref[...]
    colr = lax.broadcasted_iota(jnp.int32, (B, VPAD), 1)
    rowb = lax.broadcasted_iota(jnp.int32, (B, VPAD), 0)

    c = ps
    sh = 1
    while sh < VPAD:
        c = c + jnp.where(colr >= sh, pltpu.roll(c, sh, axis=1), 0.0)
        sh *= 2

    v = jnp.where(c <= TOP_P, ps, 0.0) * eg_ref[...]
    vmax = jnp.max(jnp.max(v, axis=1, keepdims=True), axis=0, keepdims=True)
    lin = rowb * VOCAB + colr
    cand = jnp.where(v == vmax, lin, jnp.int32(2**30))
    lin_star = jnp.min(jnp.min(cand, axis=1, keepdims=True),
                       axis=0, keepdims=True)
    b_star = lin_star // VOCAB
    r_star = lin_star - b_star * VOCAB

    p_star = jnp.sum(jnp.sum(jnp.where(lin == lin_star, ps, 0.0),
                             axis=1, keepdims=True), axis=0, keepdims=True)

    rowmask = rowb == b_star
    gt = rowmask & (p > p_star)
    cnt_gt = jnp.sum(jnp.sum(jnp.where(gt, 1, 0), axis=1, keepdims=True),
                     axis=0, keepdims=True)
    tie_pos = r_star - cnt_gt
    eq = rowmask & (p == p_star)
    eq_i = jnp.where(eq, 1, 0)
    ec = eq_i
    sh = 1
    while sh < VPAD:
        ec = ec + jnp.where(colr >= sh, pltpu.roll(ec, sh, axis=1), 0)
        sh *= 2
    win = eq & ((ec - eq_i) == tie_pos)
    tok = jnp.sum(jnp.sum(jnp.where(win, colr, 0), axis=1, keepdims=True),
                  axis=0, keepdims=True)
    out_ref[0, 0] = tok[0, 0]


@jax.jit
def kernel(inputs, W, b):
    wp = jnp.zeros((D_MODEL, VPAD), jnp.float32).at[:, :VOCAB].set(W)
    bp = jnp.full((1, VPAD), NEG, jnp.float32).at[0, :VOCAB].set(b)

    g = jax.random.gumbel(jax.random.key(1234), (B * VOCAB,),
                          dtype=jnp.float32)
    eg = jnp.zeros((B, VPAD), jnp.float32).at[:, :VOCAB].set(
        jnp.exp(g).reshape(B, VOCAB))

    p = pl.pallas_call(
        _tc_softmax_body,
        out_shape=jax.ShapeDtypeStruct((B, VPAD), jnp.float32),
    )(inputs, wp, bp)

    sort_kernel = functools.partial(
        pl.kernel,
        out_type=jax.ShapeDtypeStruct((B, VPAD), jnp.float32),
        mesh=plsc.VectorSubcoreMesh(core_axis_name="c",
                                    subcore_axis_name="s",
                                    num_cores=2, num_subcores=16),
        scratch_types=[pltpu.VMEM((VPAD,), jnp.float32)
                       for _ in range(ROWS)],
        compiler_params=pltpu.CompilerParams(needs_layout_passes=False),
    )(_sc_sort_body)
    ps = sort_kernel(p)

    tok = pl.pallas_call(
        _tc_finish_body,
        out_shape=jax.ShapeDtypeStruct((1, 1), jnp.int32),
        out_specs=pl.BlockSpec(memory_space=pltpu.SMEM),
    )(p, ps, eg)
    return tok[0, 0]


# speedup vs baseline: 2.0432x; 2.0432x over previous
import jax
import jax.numpy as jnp
from jax import lax
from jax.experimental import pallas as pl
from jax.experimental.pallas import tpu as pltpu

B = 128
D_MODEL = 1024
VOCAB = 1000
VPAD = 1024
TOP_P = 0.9
NEG = -1e30


def _body(wt_ref, xt_ref, b_ref, eg_ref, out_ref):
    logits = jnp.dot(wt_ref[...], xt_ref[...],
                     preferred_element_type=jnp.float32)
    logits = logits + b_ref[...]

    m = jnp.max(logits, axis=0, keepdims=True)
    e = jnp.exp(logits - m)
    s = jnp.sum(e, axis=0, keepdims=True)
    p_orig = e * (1.0 / s)

    row = lax.broadcasted_iota(jnp.int32, (VPAD, B), 0)
    col = lax.broadcasted_iota(jnp.int32, (VPAD, B), 1)

    bit = {}
    m = 1
    while m <= VPAD:
        bit[m] = (row & m) == 0
        m *= 2
    p = p_orig
    k = 2
    while k <= VPAD:
        j = k // 2
        while j >= 1:
            is_lo = bit[j]
            p_dn = pltpu.roll(p, VPAD - j, axis=0)
            p_up = pltpu.roll(p, j, axis=0)
            pp = jnp.where(is_lo, p_dn, p_up)
            take_max = is_lo == bit[k] if k <= VPAD // 2 else is_lo
            p = jnp.where(take_max, jnp.maximum(p, pp), jnp.minimum(p, pp))
            j //= 2
        k *= 2

    c = p
    sh = 1
    while sh < VPAD:
        c = c + jnp.where(row >= sh, pltpu.roll(c, sh, axis=0), 0.0)
        sh *= 2

    v = jnp.where(c <= TOP_P, p, 0.0) * eg_ref[...]
    vmax = jnp.max(jnp.max(v, axis=0, keepdims=True), axis=1, keepdims=True)
    lin = row * B + col
    cand = jnp.where(v == vmax, lin, jnp.int32(2**30))
    lin_star = jnp.min(jnp.min(cand, axis=0, keepdims=True),
                       axis=1, keepdims=True)
    r_star = lin_star // B
    b_star = lin_star - r_star * B

    p_star = jnp.sum(jnp.sum(jnp.where(lin == lin_star, p, 0.0),
                             axis=0, keepdims=True), axis=1, keepdims=True)

    colmask = col == b_star
    gt = colmask & (p_orig > p_star)
    cnt_gt = jnp.sum(jnp.sum(jnp.where(gt, 1, 0), axis=0, keepdims=True),
                     axis=1, keepdims=True)
    tie_pos = r_star - cnt_gt
    eq = colmask & (p_orig == p_star)
    eq_i = jnp.where(eq, 1, 0)
    ec = eq_i
    sh = 1
    while sh < VPAD:
        ec = ec + jnp.where(row >= sh, pltpu.roll(ec, sh, axis=0), 0)
        sh *= 2
    win = eq & ((ec - eq_i) == tie_pos)
    tok = jnp.sum(jnp.sum(jnp.where(win, row, 0), axis=0, keepdims=True),
                  axis=1, keepdims=True)
    out_ref[0, 0] = tok[0, 0]


@jax.jit
def kernel(inputs, W, b):
    xt = inputs.T
    wt = jnp.zeros((VPAD, D_MODEL), jnp.float32).at[:VOCAB].set(W.T)
    bp = jnp.full((VPAD, 1), NEG, jnp.float32).at[:VOCAB, 0].set(b)

    g = jax.random.gumbel(jax.random.key(1234), (B * VOCAB,),
                          dtype=jnp.float32)
    eg = jnp.zeros((VPAD, B), jnp.float32).at[:VOCAB].set(
        jnp.exp(g).reshape(B, VOCAB).T)

    tok = pl.pallas_call(
        _body,
        out_shape=jax.ShapeDtypeStruct((1, 1), jnp.int32),
        out_specs=pl.BlockSpec(memory_space=pltpu.SMEM),
    )(wt, xt, bp, eg)
    return tok[0, 0]
